# baseline (speedup 1.0000x reference)
import functools

import jax
import jax.numpy as jnp
from jax import lax
from jax.experimental import pallas as pl
from jax.experimental.pallas import tpu as pltpu
from jax.experimental.pallas import tpu_sc as plsc

_N = 10000
_E = 320000
_D = 128
_G = 64

_NC = 2
_NS = 16
_NW = _NC * _NS
_EPW = _E // _NW
_C = 128
_NCH = 80
_EPAD = _NCH * _C
_DEAD = _N
_AGG_ROWS = 16 * 632
_DEG_LEN = 16 * 640
_DEG_OUT = 10240

_mesh = plsc.VectorSubcoreMesh(core_axis_name="c", subcore_axis_name="s")


@functools.partial(
    pl.kernel,
    mesh=_mesh,
    out_type=(
        jax.ShapeDtypeStruct((_NC, _N, _D), jnp.float32),
        jax.ShapeDtypeStruct((_NC, _DEG_OUT), jnp.float32),
    ),
    scratch_types=[
        pltpu.VMEM((_NCH, _C), jnp.int32),
        pltpu.VMEM((2, _C), jnp.int32),
        pltpu.VMEM((2, _C), jnp.int32),
        pltpu.VMEM((_C, _D), jnp.float32),
        pltpu.VMEM((_C, _D), jnp.float32),
        pltpu.VMEM((_C,), jnp.float32),
        pltpu.VMEM_SHARED((_AGG_ROWS, _D), jnp.float32),
        pltpu.VMEM_SHARED((_DEG_LEN,), jnp.float32),
        pltpu.SemaphoreType.DMA,
        pltpu.SemaphoreType.DMA,
    ],
)
def _edge_aggregate_sc(x_hbm, pidx_hbm, z2_hbm, z1_hbm, agg_out, deg_out,
                       pidx, sidx, didx, rows_a, rows_b, ones_v,
                       agg_sh, deg_sh, gsem_a, gsem_b):
    cid = lax.axis_index("c")
    sid = lax.axis_index("s")
    wid = cid * _NS + sid

    pltpu.sync_copy(pidx_hbm.at[wid], pidx)
    pltpu.sync_copy(z2_hbm.at[pl.ds(sid * 632, 632)],
                    agg_sh.at[pl.ds(sid * 632, 632)])
    pltpu.sync_copy(z1_hbm.at[pl.ds(sid * 640, 640)],
                    deg_sh.at[pl.ds(sid * 640, 640)])
    one16 = jnp.ones((16,), jnp.float32)
    for k in range(_C // 16):
        ones_v[pl.ds(k * 16, 16)] = one16
    plsc.subcore_barrier()

    def unpack(j, b):
        for k in range(_C // 16):
            p = pidx[j, pl.ds(k * 16, 16)]
            sidx[b, pl.ds(k * 16, 16)] = p & 0xFFFF
            didx[b, pl.ds(k * 16, 16)] = p >> 16

    bufs = (rows_a, rows_b)
    gsems = (gsem_a, gsem_b)

    unpack(0, 0)
    pltpu.async_copy(x_hbm.at[sidx.at[0]], rows_a, gsem_a)
    unpack(1, 1)
    pltpu.async_copy(x_hbm.at[sidx.at[1]], rows_b, gsem_b)

    def chunk_pair(jj, carry):
        for b in range(2):
            j = 2 * jj + b
            j2 = jnp.minimum(j + 2, _NCH - 1)
            pltpu.make_async_copy(x_hbm.at[sidx.at[b]], bufs[b],
                                  gsems[b]).wait()
            pltpu.sync_copy(bufs[b], agg_sh.at[didx.at[b]], add=True)
            pltpu.sync_copy(ones_v, deg_sh.at[didx.at[b]], add=True)
            unpack(j2, b)
            pltpu.async_copy(x_hbm.at[sidx.at[b]], bufs[b], gsems[b])
        return carry

    lax.fori_loop(0, _NCH // 2, chunk_pair, 0)
    pltpu.make_async_copy(x_hbm.at[sidx.at[0]], rows_a, gsem_a).wait()
    pltpu.make_async_copy(x_hbm.at[sidx.at[1]], rows_b, gsem_b).wait()
    plsc.subcore_barrier()

    pltpu.sync_copy(agg_sh.at[pl.ds(sid * 624, 624)],
                    agg_out.at[cid, pl.ds(sid * 624, 624)])

    @pl.when(sid == 15)
    def _():
        pltpu.sync_copy(agg_sh.at[pl.ds(9984, 16)],
                        agg_out.at[cid, pl.ds(9984, 16)])

    @pl.when(sid < 10)
    def _():
        pltpu.sync_copy(deg_sh.at[pl.ds(sid * 1024, 1024)],
                        deg_out.at[cid, pl.ds(sid * 1024, 1024)])


_NB = 1000
_NBLK = _N // _NB


def _dense_tc(agg_ref, deg_ref, batch_ref, Wm_ref, bm_ref, Wo_ref, bo_ref,
              out_ref, sums_acc, counts_acc):
    i = pl.program_id(0)

    @pl.when(i == 0)
    def _():
        sums_acc[...] = jnp.zeros_like(sums_acc)
        counts_acc[...] = jnp.zeros_like(counts_acc)

    agg = agg_ref[0] + agg_ref[1]
    deg = deg_ref[0, 0, 0, :] + deg_ref[1, 0, 0, :]
    scale = 1.0 / jnp.maximum(deg, 1.0)
    nodes = jnp.maximum(
        (agg * scale[:, None]) @ Wm_ref[...] + bm_ref[...], 0.0)
    b = batch_ref[0, 0, :]
    onehot = (b[:, None] == lax.broadcasted_iota(jnp.int32, (1, _G), 1)
              ).astype(jnp.float32)
    sums_acc[...] += lax.dot_general(
        onehot, nodes, (((0,), (0,)), ((), ())),
        preferred_element_type=jnp.float32)
    counts_acc[...] += lax.dot_general(
        onehot, jnp.ones((_NB, 1), jnp.float32), (((0,), (0,)), ((), ())),
        preferred_element_type=jnp.float32)

    @pl.when(i == _NBLK - 1)
    def _():
        pooled = sums_acc[...] / jnp.maximum(counts_acc[...], 1.0)
        out_ref[...] = (jnp.dot(pooled, Wo_ref[...],
                                preferred_element_type=jnp.float32)
                        + bo_ref[...])


def kernel(x, edge_index, batch, W_msg, b_msg, W_out, b_out):
    src = edge_index[0].reshape(_NW, _EPW)
    dst = edge_index[1].reshape(_NW, _EPW)
    src_p = jnp.pad(src, ((0, 0), (0, _EPAD - _EPW)))
    dst_p = jnp.pad(dst, ((0, 0), (0, _EPAD - _EPW)), constant_values=_DEAD)
    packed = (src_p | (dst_p << 16)).reshape(_NW, _NCH, _C)
    z2 = jnp.zeros((_AGG_ROWS, _D), jnp.float32)
    z1 = jnp.zeros((_DEG_LEN,), jnp.float32)
    agg_p, deg_p = _edge_aggregate_sc(x, packed, z2, z1)

    out = pl.pallas_call(
        _dense_tc,
        grid=(_NBLK,),
        in_specs=[
            pl.BlockSpec((_NC, _NB, _D), lambda i: (0, i, 0)),
            pl.BlockSpec((_NC, 1, 1, _NB), lambda i: (0, i, 0, 0)),
            pl.BlockSpec((1, 1, _NB), lambda i: (i, 0, 0)),
            pl.BlockSpec((_D, _D), lambda i: (0, 0)),
            pl.BlockSpec((1, _D), lambda i: (0, 0)),
            pl.BlockSpec((_D, 1), lambda i: (0, 0)),
            pl.BlockSpec((1, 1), lambda i: (0, 0)),
        ],
        out_specs=pl.BlockSpec((_G, 1), lambda i: (0, 0)),
        out_shape=jax.ShapeDtypeStruct((_G, 1), jnp.float32),
        scratch_shapes=[
            pltpu.VMEM((_G, _D), jnp.float32),
            pltpu.VMEM((_G, 1), jnp.float32),
        ],
    )(agg_p, deg_p[:, :_N].reshape(_NC, _NBLK, 1, _NB), batch.reshape(_NBLK, 1, _NB),
      W_msg, b_msg.reshape(1, _D), W_out, b_out.reshape(1, 1))
    return out.reshape(-1)


# speedup vs baseline: 1.5366x; 1.5366x over previous
import functools

import jax
import jax.numpy as jnp
from jax import lax
from jax.experimental import pallas as pl
from jax.experimental.pallas import tpu as pltpu
from jax.experimental.pallas import tpu_sc as plsc

_N = 10000
_E = 320000
_D = 128
_G = 64

_NC = 2
_NS = 16
_NW = _NC * _NS
_EPW = _E // _NW
_C = 256
_NCH = 40
_EPAD = _NCH * _C
_DEAD = _N
_AGG_ROWS = 16 * 632
_DEG_LEN = 16 * 640
_DEG_OUT = 10240

_mesh = plsc.VectorSubcoreMesh(core_axis_name="c", subcore_axis_name="s")


@functools.partial(
    pl.kernel,
    mesh=_mesh,
    out_type=(
        jax.ShapeDtypeStruct((_NC, _N, _D), jnp.float32),
        jax.ShapeDtypeStruct((_NC, _DEG_OUT), jnp.float32),
    ),
    scratch_types=[
        pltpu.VMEM((_NCH, _C), jnp.int32),
        pltpu.VMEM((_C,), jnp.int32),
        pltpu.VMEM((_C,), jnp.int32),
        pltpu.VMEM((_C, _D), jnp.float32),
        pltpu.VMEM((_C,), jnp.float32),
        pltpu.VMEM_SHARED((_AGG_ROWS, _D), jnp.float32),
        pltpu.VMEM_SHARED((_DEG_LEN,), jnp.float32),
        pltpu.SemaphoreType.DMA,
    ],
)
def _edge_aggregate_sc(x_hbm, pidx_hbm, z2_hbm, z1_hbm, agg_out, deg_out,
                       pidx, sidx, didx, rows_v, ones_v,
                       agg_sh, deg_sh, gsem):
    cid = lax.axis_index("c")
    sid = lax.axis_index("s")
    wid = cid * _NS + sid

    pltpu.sync_copy(pidx_hbm.at[wid], pidx)
    pltpu.sync_copy(z2_hbm.at[pl.ds(sid * 632, 632)],
                    agg_sh.at[pl.ds(sid * 632, 632)])
    pltpu.sync_copy(z1_hbm.at[pl.ds(sid * 640, 640)],
                    deg_sh.at[pl.ds(sid * 640, 640)])
    one16 = jnp.ones((16,), jnp.float32)
    for k in range(_C // 16):
        ones_v[pl.ds(k * 16, 16)] = one16
    plsc.subcore_barrier()

    def unpack(j):
        for k in range(_C // 16):
            p = pidx[j, pl.ds(k * 16, 16)]
            sidx[pl.ds(k * 16, 16)] = p & 0xFFFF
            didx[pl.ds(k * 16, 16)] = p >> 16

    def chunk(j, carry):
        unpack(j)
        pltpu.async_copy(x_hbm.at[sidx], rows_v, gsem).wait()
        pltpu.sync_copy(rows_v, agg_sh.at[didx], add=True)
        pltpu.sync_copy(ones_v, deg_sh.at[didx], add=True)
        return carry

    lax.fori_loop(0, _NCH, chunk, 0)
    plsc.subcore_barrier()

    pltpu.sync_copy(agg_sh.at[pl.ds(sid * 624, 624)],
                    agg_out.at[cid, pl.ds(sid * 624, 624)])

    @pl.when(sid == 15)
    def _():
        pltpu.sync_copy(agg_sh.at[pl.ds(9984, 16)],
                        agg_out.at[cid, pl.ds(9984, 16)])

    @pl.when(sid < 10)
    def _():
        pltpu.sync_copy(deg_sh.at[pl.ds(sid * 1024, 1024)],
                        deg_out.at[cid, pl.ds(sid * 1024, 1024)])


_NB = 1000
_NBLK = _N // _NB


def _dense_tc(agg_ref, deg_ref, batch_ref, Wm_ref, bm_ref, Wo_ref, bo_ref,
              out_ref, sums_acc, counts_acc):
    i = pl.program_id(0)

    @pl.when(i == 0)
    def _():
        sums_acc[...] = jnp.zeros_like(sums_acc)
        counts_acc[...] = jnp.zeros_like(counts_acc)

    agg = agg_ref[0] + agg_ref[1]
    deg = deg_ref[0, 0, 0, :] + deg_ref[1, 0, 0, :]
    scale = 1.0 / jnp.maximum(deg, 1.0)
    nodes = jnp.maximum(
        (agg * scale[:, None]) @ Wm_ref[...] + bm_ref[...], 0.0)
    b = batch_ref[0, 0, :]
    onehot = (b[:, None] == lax.broadcasted_iota(jnp.int32, (1, _G), 1)
              ).astype(jnp.float32)
    sums_acc[...] += lax.dot_general(
        onehot, nodes, (((0,), (0,)), ((), ())),
        preferred_element_type=jnp.float32)
    counts_acc[...] += lax.dot_general(
        onehot, jnp.ones((_NB, 1), jnp.float32), (((0,), (0,)), ((), ())),
        preferred_element_type=jnp.float32)

    @pl.when(i == _NBLK - 1)
    def _():
        pooled = sums_acc[...] / jnp.maximum(counts_acc[...], 1.0)
        out_ref[...] = (jnp.dot(pooled, Wo_ref[...],
                                preferred_element_type=jnp.float32)
                        + bo_ref[...])


def kernel(x, edge_index, batch, W_msg, b_msg, W_out, b_out):
    src = edge_index[0].reshape(_NW, _EPW)
    dst = edge_index[1].reshape(_NW, _EPW)
    src_p = jnp.pad(src, ((0, 0), (0, _EPAD - _EPW)))
    dst_p = jnp.pad(dst, ((0, 0), (0, _EPAD - _EPW)), constant_values=_DEAD)
    packed = (src_p | (dst_p << 16)).reshape(_NW, _NCH, _C)
    z2 = jnp.zeros((_AGG_ROWS, _D), jnp.float32)
    z1 = jnp.zeros((_DEG_LEN,), jnp.float32)
    agg_p, deg_p = _edge_aggregate_sc(x, packed, z2, z1)

    out = pl.pallas_call(
        _dense_tc,
        grid=(_NBLK,),
        in_specs=[
            pl.BlockSpec((_NC, _NB, _D), lambda i: (0, i, 0)),
            pl.BlockSpec((_NC, 1, 1, _NB), lambda i: (0, i, 0, 0)),
            pl.BlockSpec((1, 1, _NB), lambda i: (i, 0, 0)),
            pl.BlockSpec((_D, _D), lambda i: (0, 0)),
            pl.BlockSpec((1, _D), lambda i: (0, 0)),
            pl.BlockSpec((_D, 1), lambda i: (0, 0)),
            pl.BlockSpec((1, 1), lambda i: (0, 0)),
        ],
        out_specs=pl.BlockSpec((_G, 1), lambda i: (0, 0)),
        out_shape=jax.ShapeDtypeStruct((_G, 1), jnp.float32),
        scratch_shapes=[
            pltpu.VMEM((_G, _D), jnp.float32),
            pltpu.VMEM((_G, 1), jnp.float32),
        ],
    )(agg_p, deg_p[:, :_N].reshape(_NC, _NBLK, 1, _NB), batch.reshape(_NBLK, 1, _NB),
      W_msg, b_msg.reshape(1, _D), W_out, b_out.reshape(1, 1))
    return out.reshape(-1)


# speedup vs baseline: 1.6750x; 1.0900x over previous
import functools

import jax
import jax.numpy as jnp
from jax import lax
from jax.experimental import pallas as pl
from jax.experimental.pallas import tpu as pltpu
from jax.experimental.pallas import tpu_sc as plsc

_N = 10000
_E = 320000
_D = 128
_G = 64

_NC = 2
_NS = 16
_NW = _NC * _NS
_EPW = _E // _NW
_C = 128
_NCH = 80
_HCH = _NCH // 2
_HPAIR = _HCH // 2
_EPAD = _NCH * _C
_DEAD = _N
_AGG_ROWS = 16 * 632
_DEG_LEN = 16 * 640
_DEG_OUT = 10240

_mesh = plsc.VectorSubcoreMesh(core_axis_name="c", subcore_axis_name="s")


@functools.partial(
    pl.kernel,
    mesh=_mesh,
    out_type=(
        jax.ShapeDtypeStruct((_NC, _N, _D), jnp.float32),
        jax.ShapeDtypeStruct((_NC, _DEG_OUT), jnp.float32),
    ),
    scratch_types=[
        pltpu.VMEM((_HCH, _C), jnp.int32),
        pltpu.VMEM((_HCH, _C), jnp.int32),
        pltpu.VMEM((_C, _D), jnp.float32),
        pltpu.VMEM((_C, _D), jnp.float32),
        pltpu.VMEM((_C,), jnp.float32),
        pltpu.VMEM_SHARED((_AGG_ROWS, _D), jnp.float32),
        pltpu.VMEM_SHARED((_DEG_LEN,), jnp.float32),
        pltpu.SemaphoreType.DMA,
        pltpu.SemaphoreType.DMA,
    ],
)
def _edge_aggregate_sc(x_hbm, src_hbm, dst_hbm, z2_hbm, z1_hbm, ones_hbm,
                       agg_out, deg_out,
                       sidx, didx, rows0, rows1, ones_v,
                       agg_sh, deg_sh, sem0, sem1):
    cid = lax.axis_index("c")
    sid = lax.axis_index("s")
    wid = cid * _NS + sid

    pltpu.sync_copy(ones_hbm, ones_v)
    pltpu.sync_copy(z2_hbm.at[pl.ds(sid * 632, 632)],
                    agg_sh.at[pl.ds(sid * 632, 632)])
    pltpu.sync_copy(z1_hbm.at[pl.ds(sid * 640, 640)],
                    deg_sh.at[pl.ds(sid * 640, 640)])
    plsc.subcore_barrier()

    def pair(p, carry):
        j0 = 2 * p
        j1 = j0 + 1
        pltpu.async_copy(x_hbm.at[sidx.at[j1]], rows1, sem1)
        pltpu.make_async_copy(x_hbm.at[sidx.at[j0]], rows0, sem0).wait()
        pltpu.sync_copy(rows0, agg_sh.at[didx.at[j0]], add=True)
        pltpu.sync_copy(ones_v, deg_sh.at[didx.at[j0]], add=True)

        @pl.when(p < _HPAIR - 1)
        def _():
            pltpu.async_copy(x_hbm.at[sidx.at[j0 + 2]], rows0, sem0)

        pltpu.make_async_copy(x_hbm.at[sidx.at[j1]], rows1, sem1).wait()
        pltpu.sync_copy(rows1, agg_sh.at[didx.at[j1]], add=True)
        pltpu.sync_copy(ones_v, deg_sh.at[didx.at[j1]], add=True)
        return carry

    for h in range(_NCH // _HCH):
        pltpu.sync_copy(src_hbm.at[wid, pl.ds(h * _HCH, _HCH)], sidx)
        pltpu.sync_copy(dst_hbm.at[wid, pl.ds(h * _HCH, _HCH)], didx)
        pltpu.async_copy(x_hbm.at[sidx.at[0]], rows0, sem0)
        lax.fori_loop(0, _HPAIR, pair, 0)

    plsc.subcore_barrier()

    pltpu.sync_copy(agg_sh.at[pl.ds(sid * 624, 624)],
                    agg_out.at[cid, pl.ds(sid * 624, 624)])

    @pl.when(sid == 15)
    def _():
        pltpu.sync_copy(agg_sh.at[pl.ds(9984, 16)],
                        agg_out.at[cid, pl.ds(9984, 16)])

    @pl.when(sid < 10)
    def _():
        pltpu.sync_copy(deg_sh.at[pl.ds(sid * 1024, 1024)],
                        deg_out.at[cid, pl.ds(sid * 1024, 1024)])


_NB = 1000
_NBLK = _N // _NB


def _dense_tc(agg_ref, deg_ref, batch_ref, Wm_ref, bm_ref, Wo_ref, bo_ref,
              out_ref, sums_acc, counts_acc):
    i = pl.program_id(0)

    @pl.when(i == 0)
    def _():
        sums_acc[...] = jnp.zeros_like(sums_acc)
        counts_acc[...] = jnp.zeros_like(counts_acc)

    agg = agg_ref[0] + agg_ref[1]
    deg = deg_ref[0, 0, 0, :] + deg_ref[1, 0, 0, :]
    scale = 1.0 / jnp.maximum(deg, 1.0)
    nodes = jnp.maximum(
        (agg * scale[:, None]) @ Wm_ref[...] + bm_ref[...], 0.0)
    b = batch_ref[0, 0, :]
    onehot = (b[:, None] == lax.broadcasted_iota(jnp.int32, (1, _G), 1)
              ).astype(jnp.float32)
    sums_acc[...] += lax.dot_general(
        onehot, nodes, (((0,), (0,)), ((), ())),
        preferred_element_type=jnp.float32)
    counts_acc[...] += lax.dot_general(
        onehot, jnp.ones((_NB, 1), jnp.float32), (((0,), (0,)), ((), ())),
        preferred_element_type=jnp.float32)

    @pl.when(i == _NBLK - 1)
    def _():
        pooled = sums_acc[...] / jnp.maximum(counts_acc[...], 1.0)
        out_ref[...] = (jnp.dot(pooled, Wo_ref[...],
                                preferred_element_type=jnp.float32)
                        + bo_ref[...])


def kernel(x, edge_index, batch, W_msg, b_msg, W_out, b_out):
    src = edge_index[0].reshape(_NW, _EPW)
    dst = edge_index[1].reshape(_NW, _EPW)
    src_p = jnp.pad(src, ((0, 0), (0, _EPAD - _EPW))).reshape(_NW, _NCH, _C)
    dst_p = jnp.pad(dst, ((0, 0), (0, _EPAD - _EPW)),
                    constant_values=_DEAD).reshape(_NW, _NCH, _C)
    z2 = jnp.zeros((_AGG_ROWS, _D), jnp.float32)
    z1 = jnp.zeros((_DEG_LEN,), jnp.float32)
    ones = jnp.ones((_C,), jnp.float32)
    agg_p, deg_p = _edge_aggregate_sc(x, src_p, dst_p, z2, z1, ones)

    out = pl.pallas_call(
        _dense_tc,
        grid=(_NBLK,),
        in_specs=[
            pl.BlockSpec((_NC, _NB, _D), lambda i: (0, i, 0)),
            pl.BlockSpec((_NC, 1, 1, _NB), lambda i: (0, i, 0, 0)),
            pl.BlockSpec((1, 1, _NB), lambda i: (i, 0, 0)),
            pl.BlockSpec((_D, _D), lambda i: (0, 0)),
            pl.BlockSpec((1, _D), lambda i: (0, 0)),
            pl.BlockSpec((_D, 1), lambda i: (0, 0)),
            pl.BlockSpec((1, 1), lambda i: (0, 0)),
        ],
        out_specs=pl.BlockSpec((_G, 1), lambda i: (0, 0)),
        out_shape=jax.ShapeDtypeStruct((_G, 1), jnp.float32),
        scratch_shapes=[
            pltpu.VMEM((_G, _D), jnp.float32),
            pltpu.VMEM((_G, 1), jnp.float32),
        ],
    )(agg_p, deg_p[:, :_N].reshape(_NC, _NBLK, 1, _NB), batch.reshape(_NBLK, 1, _NB),
      W_msg, b_msg.reshape(1, _D), W_out, b_out.reshape(1, 1))
    return out.reshape(-1)


# speedup vs baseline: 2.0910x; 1.2484x over previous
import functools

import jax
import jax.numpy as jnp
from jax import lax
from jax.experimental import pallas as pl
from jax.experimental.pallas import tpu as pltpu
from jax.experimental.pallas import tpu_sc as plsc

_N = 10000
_E = 320000
_D = 128
_G = 64

_NC = 2
_NS = 16
_NW = _NC * _NS
_EPW = _E // _NW
_C = 128
_NCH = 79
_EPAD = _NCH * _C
_DEAD = _N
_AGG_ROWS = 16 * 632
_DEG_LEN = 16 * 640
_DEG_OUT = 10240

_mesh = plsc.VectorSubcoreMesh(core_axis_name="c", subcore_axis_name="s")


@functools.partial(
    pl.kernel,
    mesh=_mesh,
    out_type=(
        jax.ShapeDtypeStruct((_NC, _N, _D), jnp.float32),
        jax.ShapeDtypeStruct((_NC, _DEG_OUT), jnp.float32),
    ),
    scratch_types=[
        pltpu.VMEM((_NCH, _C), jnp.int32),
        pltpu.VMEM((_NCH, _C), jnp.int32),
        pltpu.VMEM((_C, _D), jnp.float32),
        pltpu.VMEM((_C,), jnp.float32),
        pltpu.VMEM_SHARED((_AGG_ROWS, _D), jnp.float32),
        pltpu.VMEM_SHARED((_DEG_LEN,), jnp.float32),
        pltpu.SemaphoreType.DMA,
    ],
)
def _edge_aggregate_sc(x_hbm, src_hbm, dst_hbm, z2_hbm, z1_hbm, ones_hbm,
                       agg_out, deg_out,
                       sidx, didx, rows0, ones_v,
                       agg_sh, deg_sh, sem0):
    cid = lax.axis_index("c")
    sid = lax.axis_index("s")
    wid = cid * _NS + sid

    pltpu.sync_copy(src_hbm.at[wid], sidx)
    pltpu.sync_copy(dst_hbm.at[wid], didx)
    pltpu.sync_copy(ones_hbm, ones_v)
    pltpu.sync_copy(z2_hbm.at[pl.ds(sid * 632, 632)],
                    agg_sh.at[pl.ds(sid * 632, 632)])
    pltpu.sync_copy(z1_hbm.at[pl.ds(sid * 640, 640)],
                    deg_sh.at[pl.ds(sid * 640, 640)])
    plsc.subcore_barrier()

    def chunk(j, carry):
        pltpu.async_copy(x_hbm.at[sidx.at[j]], rows0, sem0).wait()
        pltpu.sync_copy(rows0, agg_sh.at[didx.at[j]], add=True)
        pltpu.sync_copy(ones_v, deg_sh.at[didx.at[j]], add=True)
        return carry

    lax.fori_loop(0, _NCH, chunk, 0)
    plsc.subcore_barrier()

    pltpu.sync_copy(agg_sh.at[pl.ds(sid * 624, 624)],
                    agg_out.at[cid, pl.ds(sid * 624, 624)])

    @pl.when(sid == 15)
    def _():
        pltpu.sync_copy(agg_sh.at[pl.ds(9984, 16)],
                        agg_out.at[cid, pl.ds(9984, 16)])

    @pl.when(sid < 10)
    def _():
        pltpu.sync_copy(deg_sh.at[pl.ds(sid * 1024, 1024)],
                        deg_out.at[cid, pl.ds(sid * 1024, 1024)])


_NB = 1000
_NBLK = _N // _NB


def _dense_tc(agg_ref, deg_ref, batch_ref, Wm_ref, bm_ref, Wo_ref, bo_ref,
              out_ref, sums_acc, counts_acc):
    i = pl.program_id(0)

    @pl.when(i == 0)
    def _():
        sums_acc[...] = jnp.zeros_like(sums_acc)
        counts_acc[...] = jnp.zeros_like(counts_acc)

    agg = agg_ref[0] + agg_ref[1]
    deg = deg_ref[0, 0, 0, :] + deg_ref[1, 0, 0, :]
    scale = 1.0 / jnp.maximum(deg, 1.0)
    nodes = jnp.maximum(
        (agg * scale[:, None]) @ Wm_ref[...] + bm_ref[...], 0.0)
    b = batch_ref[0, 0, :]
    onehot = (b[:, None] == lax.broadcasted_iota(jnp.int32, (1, _G), 1)
              ).astype(jnp.float32)
    sums_acc[...] += lax.dot_general(
        onehot, nodes, (((0,), (0,)), ((), ())),
        preferred_element_type=jnp.float32)
    counts_acc[...] += lax.dot_general(
        onehot, jnp.ones((_NB, 1), jnp.float32), (((0,), (0,)), ((), ())),
        preferred_element_type=jnp.float32)

    @pl.when(i == _NBLK - 1)
    def _():
        pooled = sums_acc[...] / jnp.maximum(counts_acc[...], 1.0)
        out_ref[...] = (jnp.dot(pooled, Wo_ref[...],
                                preferred_element_type=jnp.float32)
                        + bo_ref[...])


def kernel(x, edge_index, batch, W_msg, b_msg, W_out, b_out):
    src = edge_index[0].reshape(_NW, _EPW)
    dst = edge_index[1].reshape(_NW, _EPW)
    src_p = jnp.pad(src, ((0, 0), (0, _EPAD - _EPW))).reshape(_NW, _NCH, _C)
    dst_p = jnp.pad(dst, ((0, 0), (0, _EPAD - _EPW)),
                    constant_values=_DEAD).reshape(_NW, _NCH, _C)
    z2 = jnp.zeros((_AGG_ROWS, _D), jnp.float32)
    z1 = jnp.zeros((_DEG_LEN,), jnp.float32)
    ones = jnp.ones((_C,), jnp.float32)
    agg_p, deg_p = _edge_aggregate_sc(x, src_p, dst_p, z2, z1, ones)

    out = pl.pallas_call(
        _dense_tc,
        grid=(_NBLK,),
        in_specs=[
            pl.BlockSpec((_NC, _NB, _D), lambda i: (0, i, 0)),
            pl.BlockSpec((_NC, 1, 1, _NB), lambda i: (0, i, 0, 0)),
            pl.BlockSpec((1, 1, _NB), lambda i: (i, 0, 0)),
            pl.BlockSpec((_D, _D), lambda i: (0, 0)),
            pl.BlockSpec((1, _D), lambda i: (0, 0)),
            pl.BlockSpec((_D, 1), lambda i: (0, 0)),
            pl.BlockSpec((1, 1), lambda i: (0, 0)),
        ],
        out_specs=pl.BlockSpec((_G, 1), lambda i: (0, 0)),
        out_shape=jax.ShapeDtypeStruct((_G, 1), jnp.float32),
        scratch_shapes=[
            pltpu.VMEM((_G, _D), jnp.float32),
            pltpu.VMEM((_G, 1), jnp.float32),
        ],
    )(agg_p, deg_p[:, :_N].reshape(_NC, _NBLK, 1, _NB), batch.reshape(_NBLK, 1, _NB),
      W_msg, b_msg.reshape(1, _D), W_out, b_out.reshape(1, 1))
    return out.reshape(-1)
